# baseline (speedup 1.0000x reference)
import functools

import jax
import jax.numpy as jnp
from jax import lax
from jax.experimental import pallas as pl
from jax.experimental.pallas import tpu as pltpu
from jax.experimental.pallas import tpu_sc as plsc

GAMMA = 12.0
BATCH = 4096
NEG = 200
DIM = 64
FUSED = 128

_info = plsc.get_sparse_core_info()
NC, NS, L = _info.num_cores, _info.num_subcores, _info.num_lanes
NW = NC * NS
BPW = BATCH // NW

_GROUP_OFFS = list(range(0, NEG - L, L)) + [NEG - L]

_BITREV = [0, 8, 4, 12, 2, 10, 6, 14, 1, 9, 5, 13, 3, 11, 7, 15]


def _score_kernel(hcol_hbm, rcol_hbm, neg_hbm, ent_hbm, rel_hbm, out_hbm,
                  hidx_v, ridx_v, hsh_v, rsh_v, hr_v, nidx_v,
                  i0_v, i1_v, t0_v, t1_v, out8_v, sem0, sem1, semh, semr):
    wid = lax.axis_index("s") * NC + lax.axis_index("c")
    base = wid * BPW
    iota = lax.iota(jnp.int32, L)

    pltpu.sync_copy(hcol_hbm.at[pl.ds(base, BPW)], hidx_v)
    pltpu.sync_copy(rcol_hbm.at[pl.ds(base, BPW)], ridx_v)
    pltpu.sync_copy(neg_hbm.at[pl.ds(base, BPW), :], nidx_v)

    for i in range(BPW // L):
        sl = pl.ds(i * L, L)
        hsh_v[sl] = jnp.right_shift(hidx_v[sl], 1)
        rsh_v[sl] = jnp.right_shift(ridx_v[sl], 1)

    ch = pltpu.async_copy(ent_hbm.at[hsh_v], t0_v.at[pl.ds(0, BPW), :], semh)
    cr = pltpu.async_copy(rel_hbm.at[rsh_v], t1_v.at[pl.ds(0, BPW), :], semr)
    ch.wait()
    cr.wait()

    def hr_body(i, carry):
        hv = hidx_v[pl.ds(i * L, L)]
        rv = ridx_v[pl.ds(i * L, L)]
        hp = (hv & 1) << 6
        rp = (rv & 1) << 6
        for j in range(L):
            b = i * L + j
            ho = hp[j]
            ro = rp[j]
            for s in range(DIM // L):
                hr_v[pl.ds(b * DIM + s * L, L)] = (
                    t0_v[b, pl.ds(ho + s * L, L)]
                    + t1_v[b, pl.ds(ro + s * L, L)])
        return carry
    lax.fori_loop(0, BPW // L, hr_body, 0)

    def prep_idx(b, i_v):
        for off in _GROUP_OFFS:
            sl = pl.ds(off, L)
            i_v[sl] = jnp.right_shift(nidx_v[b, sl], 1)

    def start_tail(b, i_v, t_v):
        prep_idx(b, i_v)
        pltpu.async_copy(ent_hbm.at[i_v.at[pl.ds(0, 128)]],
                         t_v.at[pl.ds(0, 128), :], sem0)
        pltpu.async_copy(ent_hbm.at[i_v.at[pl.ds(128, NEG - 128)]],
                         t_v.at[pl.ds(128, NEG - 128), :], sem1)

    def wait_tail(i_v, t_v):
        pltpu.make_async_copy(ent_hbm.at[i_v.at[pl.ds(0, 128)]],
                              t_v.at[pl.ds(0, 128), :], sem0).wait()
        pltpu.make_async_copy(ent_hbm.at[i_v.at[pl.ds(128, NEG - 128)]],
                              t_v.at[pl.ds(128, NEG - 128), :], sem1).wait()

    masks = {s: (iota & s) == 0 for s in (1, 2, 4, 8)}
    rollm = {s: (iota - s) % L for s in (1, 2, 4, 8)}
    rollp = {s: (iota + s) % L for s in (1, 2, 4, 8)}

    def combine(a, bb, s):
        rb = bb.at[rollm[s]].get(mode="promise_in_bounds")
        ra = a.at[rollp[s]].get(mode="promise_in_bounds")
        return jnp.where(masks[s], a, rb) + jnp.where(masks[s], ra, bb)

    def compute_b(b, t_v):
        hr = [hr_v[pl.ds(b * DIM + s * L, L)] for s in range(DIM // L)]
        brow = b & 7

        def g_body(g, carry):
            off = lax.min(g * L, NEG - L)
            par = (nidx_v[b, pl.ds(off, L)] & 1) << 6
            parts = []
            for k in _BITREV:
                row = off + k
                o = par[k]
                p01 = (jnp.abs(hr[0] - t_v[row, pl.ds(o, L)])
                       + jnp.abs(hr[1] - t_v[row, pl.ds(o + L, L)]))
                p23 = (jnp.abs(hr[2] - t_v[row, pl.ds(o + 2 * L, L)])
                       + jnp.abs(hr[3] - t_v[row, pl.ds(o + 3 * L, L)]))
                parts.append(p01 + p23)
            w = [combine(parts[2 * i], parts[2 * i + 1], 8) for i in range(8)]
            x = [combine(w[2 * i], w[2 * i + 1], 4) for i in range(4)]
            y = [combine(x[2 * i], x[2 * i + 1], 2) for i in range(2)]
            z = combine(y[0], y[1], 1)
            out8_v[brow, pl.ds(off, L)] = GAMMA - z
            return carry

        lax.fori_loop(0, len(_GROUP_OFFS), g_body, 0)

    start_tail(0, i0_v, t0_v)

    def b_body(i, carry):
        b = i * 2
        start_tail(b + 1, i1_v, t1_v)
        wait_tail(i0_v, t0_v)
        compute_b(b, t0_v)

        @pl.when(b + 2 < BPW)
        def _():
            start_tail(b + 2, i0_v, t0_v)
        wait_tail(i1_v, t1_v)
        compute_b(b + 1, t1_v)

        @pl.when((b & 7) == 6)
        def _():
            start = pl.multiple_of(base + b - 6, 8)
            pltpu.sync_copy(out8_v, out_hbm.at[pl.ds(start, 8), :])
        return carry

    lax.fori_loop(0, BPW // 2, b_body, 0)


@jax.jit
def _kge_score(positive_sample, negative_sample, entity_embedding,
               relation_embedding):
    hcol = positive_sample[:, 0]
    rcol = positive_sample[:, 1]
    ent2 = jnp.reshape(entity_embedding, (-1, FUSED))
    rel2 = jnp.reshape(relation_embedding, (-1, FUSED))
    mesh = plsc.VectorSubcoreMesh(core_axis_name="c", subcore_axis_name="s")
    run = functools.partial(
        pl.kernel,
        out_type=jax.ShapeDtypeStruct((BATCH, NEG), jnp.float32),
        mesh=mesh,
        compiler_params=pltpu.CompilerParams(
            needs_layout_passes=False, use_tc_tiling_on_sc=True),
        scratch_types=[
            pltpu.VMEM((BPW,), jnp.int32),
            pltpu.VMEM((BPW,), jnp.int32),
            pltpu.VMEM((BPW,), jnp.int32),
            pltpu.VMEM((BPW,), jnp.int32),
            pltpu.VMEM((BPW * DIM,), jnp.float32),
            pltpu.VMEM((BPW, NEG), jnp.int32),
            pltpu.VMEM((NEG,), jnp.int32),
            pltpu.VMEM((NEG,), jnp.int32),
            pltpu.VMEM((NEG, FUSED), jnp.float32),
            pltpu.VMEM((NEG, FUSED), jnp.float32),
            pltpu.VMEM((8, NEG), jnp.float32),
            pltpu.SemaphoreType.DMA,
            pltpu.SemaphoreType.DMA,
            pltpu.SemaphoreType.DMA,
            pltpu.SemaphoreType.DMA,
        ],
    )(_score_kernel)
    return run(hcol, rcol, negative_sample, ent2, rel2)


def kernel(positive_sample, negative_sample, entity_embedding,
           relation_embedding):
    return _kge_score(positive_sample, negative_sample, entity_embedding,
                      relation_embedding)


# speedup vs baseline: 1.4785x; 1.4785x over previous
import functools

import jax
import jax.numpy as jnp
from jax import lax
from jax.experimental import pallas as pl
from jax.experimental.pallas import tpu as pltpu
from jax.experimental.pallas import tpu_sc as plsc

GAMMA = 12.0
BATCH = 4096
NEG = 200
DIM = 64
FUSED = 128
EBLK = 4096
HALF = EBLK // 2

_info = plsc.get_sparse_core_info()
NC, NS, L = _info.num_cores, _info.num_subcores, _info.num_lanes
NW = NC * NS
BPW = BATCH // NW

_GROUP_OFFS = list(range(0, NEG - L, L)) + [NEG - L]

_BITREV = [0, 8, 4, 12, 2, 10, 6, 14, 1, 9, 5, 13, 3, 11, 7, 15]


def _repack_kernel(in_ref, out_ref):
    x = in_ref[...]
    out_ref[:, 0:DIM] = x[:, 0:HALF].T
    out_ref[:, DIM:FUSED] = x[:, HALF:EBLK].T


def _to_fused(table):
    n = table.shape[0]
    nblk = pl.cdiv(n, EBLK)
    return pl.pallas_call(
        _repack_kernel,
        grid=(nblk,),
        in_specs=[pl.BlockSpec((DIM, EBLK), lambda i: (0, i))],
        out_specs=pl.BlockSpec((HALF, FUSED), lambda i: (i, 0)),
        out_shape=jax.ShapeDtypeStruct((nblk * HALF, FUSED), jnp.float32),
    )(table.T)


def _fused_row(e):
    return ((e >> 12) << 11) | (e & (HALF - 1))


def _score_kernel(hcol_hbm, rcol_hbm, neg_hbm, ent_hbm, rel_hbm, out_hbm,
                  hidx_v, ridx_v, hsh_v, rsh_v, hr_v, nidx_v,
                  i0_v, i1_v, t0_v, t1_v, out8_v, sem0, sem1, semh, semr):
    wid = lax.axis_index("s") * NC + lax.axis_index("c")
    base = wid * BPW
    iota = lax.iota(jnp.int32, L)

    pltpu.sync_copy(hcol_hbm.at[pl.ds(base, BPW)], hidx_v)
    pltpu.sync_copy(rcol_hbm.at[pl.ds(base, BPW)], ridx_v)
    pltpu.sync_copy(neg_hbm.at[pl.ds(base, BPW), :], nidx_v)

    for i in range(BPW // L):
        sl = pl.ds(i * L, L)
        hsh_v[sl] = _fused_row(hidx_v[sl])
        rsh_v[sl] = _fused_row(ridx_v[sl])

    ch = pltpu.async_copy(ent_hbm.at[hsh_v], t0_v.at[pl.ds(0, BPW), :], semh)
    cr = pltpu.async_copy(rel_hbm.at[rsh_v], t1_v.at[pl.ds(0, BPW), :], semr)
    ch.wait()
    cr.wait()

    def hr_body(i, carry):
        hp = ((hidx_v[pl.ds(i * L, L)] >> 11) & 1) << 6
        rp = ((ridx_v[pl.ds(i * L, L)] >> 11) & 1) << 6
        for j in range(L):
            b = i * L + j
            ho = hp[j]
            ro = rp[j]
            for s in range(DIM // L):
                hr_v[pl.ds(b * DIM + s * L, L)] = (
                    t0_v[b, pl.ds(ho + s * L, L)]
                    + t1_v[b, pl.ds(ro + s * L, L)])
        return carry
    lax.fori_loop(0, BPW // L, hr_body, 0)

    def prep_idx(b, i_v):
        for off in _GROUP_OFFS:
            sl = pl.ds(off, L)
            i_v[sl] = _fused_row(nidx_v[b, sl])

    def start_tail(b, i_v, t_v):
        prep_idx(b, i_v)
        pltpu.async_copy(ent_hbm.at[i_v.at[pl.ds(0, 128)]],
                         t_v.at[pl.ds(0, 128), :], sem0)
        pltpu.async_copy(ent_hbm.at[i_v.at[pl.ds(128, NEG - 128)]],
                         t_v.at[pl.ds(128, NEG - 128), :], sem1)

    def wait_tail(i_v, t_v):
        pltpu.make_async_copy(ent_hbm.at[i_v.at[pl.ds(0, 128)]],
                              t_v.at[pl.ds(0, 128), :], sem0).wait()
        pltpu.make_async_copy(ent_hbm.at[i_v.at[pl.ds(128, NEG - 128)]],
                              t_v.at[pl.ds(128, NEG - 128), :], sem1).wait()

    masks = {s: (iota & s) == 0 for s in (1, 2, 4, 8)}
    rollm = {s: (iota - s) % L for s in (1, 2, 4, 8)}
    rollp = {s: (iota + s) % L for s in (1, 2, 4, 8)}

    def combine(a, bb, s):
        rb = bb.at[rollm[s]].get(mode="promise_in_bounds")
        ra = a.at[rollp[s]].get(mode="promise_in_bounds")
        return jnp.where(masks[s], a, rb) + jnp.where(masks[s], ra, bb)

    def compute_b(b, t_v):
        hr = [hr_v[pl.ds(b * DIM + s * L, L)] for s in range(DIM // L)]
        brow = b & 7

        def g_body(g, carry):
            off = lax.min(g * L, NEG - L)
            par = ((nidx_v[b, pl.ds(off, L)] >> 11) & 1) << 6
            parts = []
            for k in _BITREV:
                row = off + k
                o = par[k]
                p01 = (jnp.abs(hr[0] - t_v[row, pl.ds(o, L)])
                       + jnp.abs(hr[1] - t_v[row, pl.ds(o + L, L)]))
                p23 = (jnp.abs(hr[2] - t_v[row, pl.ds(o + 2 * L, L)])
                       + jnp.abs(hr[3] - t_v[row, pl.ds(o + 3 * L, L)]))
                parts.append(p01 + p23)
            w = [combine(parts[2 * i], parts[2 * i + 1], 8) for i in range(8)]
            x = [combine(w[2 * i], w[2 * i + 1], 4) for i in range(4)]
            y = [combine(x[2 * i], x[2 * i + 1], 2) for i in range(2)]
            z = combine(y[0], y[1], 1)
            out8_v[brow, pl.ds(off, L)] = GAMMA - z
            return carry

        lax.fori_loop(0, len(_GROUP_OFFS), g_body, 0)

    start_tail(0, i0_v, t0_v)

    def b_body(i, carry):
        b = i * 2
        start_tail(b + 1, i1_v, t1_v)
        wait_tail(i0_v, t0_v)
        compute_b(b, t0_v)

        @pl.when(b + 2 < BPW)
        def _():
            start_tail(b + 2, i0_v, t0_v)
        wait_tail(i1_v, t1_v)
        compute_b(b + 1, t1_v)

        @pl.when((b & 7) == 6)
        def _():
            start = pl.multiple_of(base + b - 6, 8)
            pltpu.sync_copy(out8_v, out_hbm.at[pl.ds(start, 8), :])
        return carry

    lax.fori_loop(0, BPW // 2, b_body, 0)


@jax.jit
def _kge_score(positive_sample, negative_sample, entity_embedding,
               relation_embedding):
    hcol = positive_sample[:, 0]
    rcol = positive_sample[:, 1]
    ent2 = _to_fused(entity_embedding)
    rel2 = _to_fused(relation_embedding)
    mesh = plsc.VectorSubcoreMesh(core_axis_name="c", subcore_axis_name="s")
    run = functools.partial(
        pl.kernel,
        out_type=jax.ShapeDtypeStruct((BATCH, NEG), jnp.float32),
        mesh=mesh,
        compiler_params=pltpu.CompilerParams(
            needs_layout_passes=False, use_tc_tiling_on_sc=True),
        scratch_types=[
            pltpu.VMEM((BPW,), jnp.int32),
            pltpu.VMEM((BPW,), jnp.int32),
            pltpu.VMEM((BPW,), jnp.int32),
            pltpu.VMEM((BPW,), jnp.int32),
            pltpu.VMEM((BPW * DIM,), jnp.float32),
            pltpu.VMEM((BPW, NEG), jnp.int32),
            pltpu.VMEM((NEG,), jnp.int32),
            pltpu.VMEM((NEG,), jnp.int32),
            pltpu.VMEM((NEG, FUSED), jnp.float32),
            pltpu.VMEM((NEG, FUSED), jnp.float32),
            pltpu.VMEM((8, NEG), jnp.float32),
            pltpu.SemaphoreType.DMA,
            pltpu.SemaphoreType.DMA,
            pltpu.SemaphoreType.DMA,
            pltpu.SemaphoreType.DMA,
        ],
    )(_score_kernel)
    return run(hcol, rcol, negative_sample, ent2, rel2)


def kernel(positive_sample, negative_sample, entity_embedding,
           relation_embedding):
    return _kge_score(positive_sample, negative_sample, entity_embedding,
                      relation_embedding)


# speedup vs baseline: 1.6281x; 1.1012x over previous
import functools

import jax
import jax.numpy as jnp
from jax import lax
from jax.experimental import pallas as pl
from jax.experimental.pallas import tpu as pltpu
from jax.experimental.pallas import tpu_sc as plsc

GAMMA = 12.0
BATCH = 4096
NEG = 200
DIM = 64
FUSED = 128
EBLK = 4096
HALF = EBLK // 2

_info = plsc.get_sparse_core_info()
NC, NS, L = _info.num_cores, _info.num_subcores, _info.num_lanes
NW = NC * NS
BPW = BATCH // NW

_GROUP_OFFS = list(range(0, NEG - L, L)) + [NEG - L]

_BITREV = [0, 8, 4, 12, 2, 10, 6, 14, 1, 9, 5, 13, 3, 11, 7, 15]


def _repack_kernel(in_ref, out_ref):
    x = in_ref[...]
    out_ref[:, 0:DIM] = x[:, 0:HALF].T
    out_ref[:, DIM:FUSED] = x[:, HALF:EBLK].T


def _to_fused(table):
    n = table.shape[0]
    nblk = pl.cdiv(n, EBLK)
    return pl.pallas_call(
        _repack_kernel,
        grid=(nblk,),
        in_specs=[pl.BlockSpec((DIM, EBLK), lambda i: (0, i))],
        out_specs=pl.BlockSpec((HALF, FUSED), lambda i: (i, 0)),
        out_shape=jax.ShapeDtypeStruct((nblk * HALF, FUSED), jnp.float32),
    )(table.T)


def _row64(e):
    return ((e >> 12) << 12) | ((e & (HALF - 1)) << 1) | ((e >> 11) & 1)


def _score_kernel(hcol_hbm, rcol_hbm, neg_hbm, ent_hbm, rel_hbm, out_hbm,
                  hidx_v, ridx_v, hsh_v, rsh_v, hr_v, nidx_v,
                  i0_v, i1_v, t0_v, t1_v, out8_v, sem0, sem1, semh, semr):
    wid = lax.axis_index("s") * NC + lax.axis_index("c")
    base = wid * BPW
    iota = lax.iota(jnp.int32, L)

    pltpu.sync_copy(hcol_hbm.at[pl.ds(base, BPW)], hidx_v)
    pltpu.sync_copy(rcol_hbm.at[pl.ds(base, BPW)], ridx_v)
    pltpu.sync_copy(neg_hbm.at[pl.ds(base, BPW), :], nidx_v)

    for i in range(BPW // L):
        sl = pl.ds(i * L, L)
        hsh_v[sl] = _row64(hidx_v[sl])
        rsh_v[sl] = _row64(ridx_v[sl])

    ch = pltpu.async_copy(ent_hbm.at[hsh_v], t0_v.at[pl.ds(0, BPW), :], semh)
    cr = pltpu.async_copy(rel_hbm.at[rsh_v], t1_v.at[pl.ds(0, BPW), :], semr)
    ch.wait()
    cr.wait()

    def hr_body(b, carry):
        for s in range(DIM // L):
            sl = pl.ds(s * L, L)
            hr_v[pl.ds(b * DIM + s * L, L)] = t0_v[b, sl] + t1_v[b, sl]
        return carry
    lax.fori_loop(0, BPW, hr_body, 0)

    def prep_idx(b, i_v):
        for off in _GROUP_OFFS:
            sl = pl.ds(off, L)
            i_v[sl] = _row64(nidx_v[b, sl])

    def start_tail(b, i_v, t_v):
        prep_idx(b, i_v)
        pltpu.async_copy(ent_hbm.at[i_v.at[pl.ds(0, 128)]],
                         t_v.at[pl.ds(0, 128), :], sem0)
        pltpu.async_copy(ent_hbm.at[i_v.at[pl.ds(128, NEG - 128)]],
                         t_v.at[pl.ds(128, NEG - 128), :], sem1)

    def wait_tail(i_v, t_v):
        pltpu.make_async_copy(ent_hbm.at[i_v.at[pl.ds(0, 128)]],
                              t_v.at[pl.ds(0, 128), :], sem0).wait()
        pltpu.make_async_copy(ent_hbm.at[i_v.at[pl.ds(128, NEG - 128)]],
                              t_v.at[pl.ds(128, NEG - 128), :], sem1).wait()

    masks = {s: (iota & s) == 0 for s in (1, 2, 4, 8)}
    rollm = {s: (iota - s) % L for s in (1, 2, 4, 8)}
    rollp = {s: (iota + s) % L for s in (1, 2, 4, 8)}

    def combine(a, bb, s):
        rb = bb.at[rollm[s]].get(mode="promise_in_bounds")
        ra = a.at[rollp[s]].get(mode="promise_in_bounds")
        return jnp.where(masks[s], a, rb) + jnp.where(masks[s], ra, bb)

    def compute_b(b, t_v):
        hr = [hr_v[pl.ds(b * DIM + s * L, L)] for s in range(DIM // L)]
        brow = b & 7

        def g_body(g, carry):
            off = lax.min(g * L, NEG - L)
            parts = []
            for k in _BITREV:
                row = off + k
                p01 = (jnp.abs(hr[0] - t_v[row, pl.ds(0, L)])
                       + jnp.abs(hr[1] - t_v[row, pl.ds(L, L)]))
                p23 = (jnp.abs(hr[2] - t_v[row, pl.ds(2 * L, L)])
                       + jnp.abs(hr[3] - t_v[row, pl.ds(3 * L, L)]))
                parts.append(p01 + p23)
            w = [combine(parts[2 * i], parts[2 * i + 1], 8) for i in range(8)]
            x = [combine(w[2 * i], w[2 * i + 1], 4) for i in range(4)]
            y = [combine(x[2 * i], x[2 * i + 1], 2) for i in range(2)]
            z = combine(y[0], y[1], 1)
            out8_v[brow, pl.ds(off, L)] = GAMMA - z
            return carry

        lax.fori_loop(0, len(_GROUP_OFFS), g_body, 0)

    start_tail(0, i0_v, t0_v)

    def b_body(i, carry):
        b = i * 2
        start_tail(b + 1, i1_v, t1_v)
        wait_tail(i0_v, t0_v)
        compute_b(b, t0_v)

        @pl.when(b + 2 < BPW)
        def _():
            start_tail(b + 2, i0_v, t0_v)
        wait_tail(i1_v, t1_v)
        compute_b(b + 1, t1_v)

        @pl.when((b & 7) == 6)
        def _():
            start = pl.multiple_of(base + b - 6, 8)
            pltpu.sync_copy(out8_v, out_hbm.at[pl.ds(start, 8), :])
        return carry

    lax.fori_loop(0, BPW // 2, b_body, 0)


@jax.jit
def _kge_score(positive_sample, negative_sample, entity_embedding,
               relation_embedding):
    hcol = positive_sample[:, 0]
    rcol = positive_sample[:, 1]
    ent2 = jnp.reshape(_to_fused(entity_embedding), (-1, DIM))
    rel2 = jnp.reshape(_to_fused(relation_embedding), (-1, DIM))
    mesh = plsc.VectorSubcoreMesh(core_axis_name="c", subcore_axis_name="s")
    run = functools.partial(
        pl.kernel,
        out_type=jax.ShapeDtypeStruct((BATCH, NEG), jnp.float32),
        mesh=mesh,
        compiler_params=pltpu.CompilerParams(
            needs_layout_passes=False, use_tc_tiling_on_sc=False),
        scratch_types=[
            pltpu.VMEM((BPW,), jnp.int32),
            pltpu.VMEM((BPW,), jnp.int32),
            pltpu.VMEM((BPW,), jnp.int32),
            pltpu.VMEM((BPW,), jnp.int32),
            pltpu.VMEM((BPW * DIM,), jnp.float32),
            pltpu.VMEM((BPW, NEG), jnp.int32),
            pltpu.VMEM((NEG,), jnp.int32),
            pltpu.VMEM((NEG,), jnp.int32),
            pltpu.VMEM((NEG, DIM), jnp.float32),
            pltpu.VMEM((NEG, DIM), jnp.float32),
            pltpu.VMEM((8, NEG), jnp.float32),
            pltpu.SemaphoreType.DMA,
            pltpu.SemaphoreType.DMA,
            pltpu.SemaphoreType.DMA,
            pltpu.SemaphoreType.DMA,
        ],
    )(_score_kernel)
    return run(hcol, rcol, negative_sample, ent2, rel2)


def kernel(positive_sample, negative_sample, entity_embedding,
           relation_embedding):
    return _kge_score(positive_sample, negative_sample, entity_embedding,
                      relation_embedding)


# speedup vs baseline: 1.8789x; 1.1541x over previous
import functools

import jax
import jax.numpy as jnp
from jax import lax
from jax.experimental import pallas as pl
from jax.experimental.pallas import tpu as pltpu
from jax.experimental.pallas import tpu_sc as plsc

GAMMA = 12.0
BATCH = 4096
NEG = 200
DIM = 64
FUSED = 128
EBLK = 8192
HALF = EBLK // 2

_info = plsc.get_sparse_core_info()
NC, NS, L = _info.num_cores, _info.num_subcores, _info.num_lanes
NW = NC * NS
BPW = BATCH // NW

_GROUP_OFFS = list(range(0, NEG - L, L)) + [NEG - L]

_BITREV = [0, 8, 4, 12, 2, 10, 6, 14, 1, 9, 5, 13, 3, 11, 7, 15]


def _repack_kernel(in_ref, out_ref):
    x = in_ref[...]
    out_ref[:, 0:DIM] = x[:, 0:HALF].T
    out_ref[:, DIM:FUSED] = x[:, HALF:EBLK].T


def _to_fused(table):
    n = table.shape[0]
    nblk = pl.cdiv(n, EBLK)
    return pl.pallas_call(
        _repack_kernel,
        grid=(nblk,),
        in_specs=[pl.BlockSpec((DIM, EBLK), lambda i: (0, i))],
        out_specs=pl.BlockSpec((HALF, FUSED), lambda i: (i, 0)),
        out_shape=jax.ShapeDtypeStruct((nblk * HALF, FUSED), jnp.float32),
    )(table.T)


_SH = EBLK.bit_length() - 1


def _row64(e):
    return ((e >> _SH) << _SH) | ((e & (HALF - 1)) << 1) | ((e >> (_SH - 1)) & 1)


def _score_kernel(hcol_hbm, rcol_hbm, neg_hbm, ent_hbm, rel_hbm, out_hbm,
                  hidx_v, ridx_v, hsh_v, rsh_v, hr_v, nidx_v,
                  i0_v, i1_v, t0_v, t1_v, out8_v, sem0, sem1, semh, semr):
    wid = lax.axis_index("s") * NC + lax.axis_index("c")
    base = wid * BPW
    iota = lax.iota(jnp.int32, L)

    pltpu.sync_copy(hcol_hbm.at[pl.ds(base, BPW)], hidx_v)
    pltpu.sync_copy(rcol_hbm.at[pl.ds(base, BPW)], ridx_v)
    pltpu.sync_copy(neg_hbm.at[pl.ds(base, BPW), :], nidx_v)

    for i in range(BPW // L):
        sl = pl.ds(i * L, L)
        hsh_v[sl] = _row64(hidx_v[sl])
        rsh_v[sl] = _row64(ridx_v[sl])

    ch = pltpu.async_copy(ent_hbm.at[hsh_v], t0_v.at[pl.ds(0, BPW), :], semh)
    cr = pltpu.async_copy(rel_hbm.at[rsh_v], t1_v.at[pl.ds(0, BPW), :], semr)
    ch.wait()
    cr.wait()

    def hr_body(b, carry):
        for s in range(DIM // L):
            sl = pl.ds(s * L, L)
            hr_v[pl.ds(b * DIM + s * L, L)] = t0_v[b, sl] + t1_v[b, sl]
        return carry
    lax.fori_loop(0, BPW, hr_body, 0)

    def prep_idx(b, i_v):
        for off in _GROUP_OFFS:
            sl = pl.ds(off, L)
            i_v[sl] = _row64(nidx_v[b, sl])

    def start_tail(b, i_v, t_v):
        prep_idx(b, i_v)
        pltpu.async_copy(ent_hbm.at[i_v.at[pl.ds(0, 128)]],
                         t_v.at[pl.ds(0, 128), :], sem0)
        pltpu.async_copy(ent_hbm.at[i_v.at[pl.ds(128, NEG - 128)]],
                         t_v.at[pl.ds(128, NEG - 128), :], sem1)

    def wait_tail(i_v, t_v):
        pltpu.make_async_copy(ent_hbm.at[i_v.at[pl.ds(0, 128)]],
                              t_v.at[pl.ds(0, 128), :], sem0).wait()
        pltpu.make_async_copy(ent_hbm.at[i_v.at[pl.ds(128, NEG - 128)]],
                              t_v.at[pl.ds(128, NEG - 128), :], sem1).wait()

    masks = {s: (iota & s) == 0 for s in (1, 2, 4, 8)}
    rollm = {s: (iota - s) % L for s in (1, 2, 4, 8)}
    rollp = {s: (iota + s) % L for s in (1, 2, 4, 8)}

    def combine(a, bb, s):
        rb = bb.at[rollm[s]].get(mode="promise_in_bounds")
        ra = a.at[rollp[s]].get(mode="promise_in_bounds")
        return jnp.where(masks[s], a, rb) + jnp.where(masks[s], ra, bb)

    def compute_b(b, t_v):
        hr = [hr_v[pl.ds(b * DIM + s * L, L)] for s in range(DIM // L)]
        brow = b & 7

        def g_body(g, carry):
            off = lax.min(g * L, NEG - L)
            parts = []
            for k in _BITREV:
                row = off + k
                p01 = (jnp.abs(hr[0] - t_v[row, pl.ds(0, L)])
                       + jnp.abs(hr[1] - t_v[row, pl.ds(L, L)]))
                p23 = (jnp.abs(hr[2] - t_v[row, pl.ds(2 * L, L)])
                       + jnp.abs(hr[3] - t_v[row, pl.ds(3 * L, L)]))
                parts.append(p01 + p23)
            w = [combine(parts[2 * i], parts[2 * i + 1], 8) for i in range(8)]
            x = [combine(w[2 * i], w[2 * i + 1], 4) for i in range(4)]
            y = [combine(x[2 * i], x[2 * i + 1], 2) for i in range(2)]
            z = combine(y[0], y[1], 1)
            out8_v[brow, pl.ds(off, L)] = GAMMA - z
            return carry

        lax.fori_loop(0, len(_GROUP_OFFS), g_body, 0)

    start_tail(0, i0_v, t0_v)

    def b_body(i, carry):
        b = i * 2
        start_tail(b + 1, i1_v, t1_v)
        wait_tail(i0_v, t0_v)
        compute_b(b, t0_v)

        @pl.when(b + 2 < BPW)
        def _():
            start_tail(b + 2, i0_v, t0_v)
        wait_tail(i1_v, t1_v)
        compute_b(b + 1, t1_v)

        @pl.when((b & 7) == 6)
        def _():
            start = pl.multiple_of(base + b - 6, 8)
            pltpu.sync_copy(out8_v, out_hbm.at[pl.ds(start, 8), :])
        return carry

    lax.fori_loop(0, BPW // 2, b_body, 0)


@jax.jit
def _kge_score(positive_sample, negative_sample, entity_embedding,
               relation_embedding):
    hcol = positive_sample[:, 0]
    rcol = positive_sample[:, 1]
    ent2 = jnp.reshape(_to_fused(entity_embedding), (-1, DIM))
    rel2 = jnp.reshape(_to_fused(relation_embedding), (-1, DIM))
    mesh = plsc.VectorSubcoreMesh(core_axis_name="c", subcore_axis_name="s")
    run = functools.partial(
        pl.kernel,
        out_type=jax.ShapeDtypeStruct((BATCH, NEG), jnp.float32),
        mesh=mesh,
        compiler_params=pltpu.CompilerParams(
            needs_layout_passes=False, use_tc_tiling_on_sc=False),
        scratch_types=[
            pltpu.VMEM((BPW,), jnp.int32),
            pltpu.VMEM((BPW,), jnp.int32),
            pltpu.VMEM((BPW,), jnp.int32),
            pltpu.VMEM((BPW,), jnp.int32),
            pltpu.VMEM((BPW * DIM,), jnp.float32),
            pltpu.VMEM((BPW, NEG), jnp.int32),
            pltpu.VMEM((NEG,), jnp.int32),
            pltpu.VMEM((NEG,), jnp.int32),
            pltpu.VMEM((NEG, DIM), jnp.float32),
            pltpu.VMEM((NEG, DIM), jnp.float32),
            pltpu.VMEM((8, NEG), jnp.float32),
            pltpu.SemaphoreType.DMA,
            pltpu.SemaphoreType.DMA,
            pltpu.SemaphoreType.DMA,
            pltpu.SemaphoreType.DMA,
        ],
    )(_score_kernel)
    return run(hcol, rcol, negative_sample, ent2, rel2)


def kernel(positive_sample, negative_sample, entity_embedding,
           relation_embedding):
    return _kge_score(positive_sample, negative_sample, entity_embedding,
                      relation_embedding)


# speedup vs baseline: 2.0101x; 1.0698x over previous
import functools

import jax
import jax.numpy as jnp
from jax import lax
from jax.experimental import pallas as pl
from jax.experimental.pallas import tpu as pltpu
from jax.experimental.pallas import tpu_sc as plsc

GAMMA = 12.0
BATCH = 4096
NEG = 200
DIM = 64
FUSED = 128
EBLK = 16384
HALF = EBLK // 2

_info = plsc.get_sparse_core_info()
NC, NS, L = _info.num_cores, _info.num_subcores, _info.num_lanes
NW = NC * NS
BPW = BATCH // NW

_GROUP_OFFS = list(range(0, NEG - L, L)) + [NEG - L]

_BITREV = [0, 8, 4, 12, 2, 10, 6, 14, 1, 9, 5, 13, 3, 11, 7, 15]


def _repack_kernel(in_ref, out_ref):
    x = in_ref[...]
    out_ref[:, 0:DIM] = x[:, 0:HALF].T
    out_ref[:, DIM:FUSED] = x[:, HALF:EBLK].T


def _to_fused(table):
    n = table.shape[0]
    nblk = pl.cdiv(n, EBLK)
    return pl.pallas_call(
        _repack_kernel,
        grid=(nblk,),
        in_specs=[pl.BlockSpec((DIM, EBLK), lambda i: (0, i))],
        out_specs=pl.BlockSpec((HALF, FUSED), lambda i: (i, 0)),
        out_shape=jax.ShapeDtypeStruct((nblk * HALF, FUSED), jnp.float32),
    )(table.T)


_SH = EBLK.bit_length() - 1


def _row64(e):
    return ((e >> _SH) << _SH) | ((e & (HALF - 1)) << 1) | ((e >> (_SH - 1)) & 1)


def _score_kernel(hcol_hbm, rcol_hbm, neg_hbm, ent_hbm, rel_hbm, out_hbm,
                  hidx_v, ridx_v, hsh_v, rsh_v, hr_v, nidx_v,
                  i0_v, i1_v, t0_v, t1_v, out8_v, sem0, sem1, semh, semr):
    wid = lax.axis_index("s") * NC + lax.axis_index("c")
    base = wid * BPW
    iota = lax.iota(jnp.int32, L)

    pltpu.sync_copy(hcol_hbm.at[pl.ds(base, BPW)], hidx_v)
    pltpu.sync_copy(rcol_hbm.at[pl.ds(base, BPW)], ridx_v)
    pltpu.sync_copy(neg_hbm.at[pl.ds(base, BPW), :], nidx_v)

    for i in range(BPW // L):
        sl = pl.ds(i * L, L)
        hsh_v[sl] = _row64(hidx_v[sl])
        rsh_v[sl] = _row64(ridx_v[sl])

    ch = pltpu.async_copy(ent_hbm.at[hsh_v], t0_v.at[pl.ds(0, BPW), :], semh)
    cr = pltpu.async_copy(rel_hbm.at[rsh_v], t1_v.at[pl.ds(0, BPW), :], semr)
    ch.wait()
    cr.wait()

    def hr_body(b, carry):
        for s in range(DIM // L):
            sl = pl.ds(s * L, L)
            hr_v[pl.ds(b * DIM + s * L, L)] = t0_v[b, sl] + t1_v[b, sl]
        return carry
    lax.fori_loop(0, BPW, hr_body, 0)

    def prep_idx(b, i_v):
        for off in _GROUP_OFFS:
            sl = pl.ds(off, L)
            i_v[sl] = _row64(nidx_v[b, sl])

    def start_tail(b, i_v, t_v):
        prep_idx(b, i_v)
        pltpu.async_copy(ent_hbm.at[i_v.at[pl.ds(0, 128)]],
                         t_v.at[pl.ds(0, 128), :], sem0)
        pltpu.async_copy(ent_hbm.at[i_v.at[pl.ds(128, NEG - 128)]],
                         t_v.at[pl.ds(128, NEG - 128), :], sem1)

    def wait_tail(i_v, t_v):
        pltpu.make_async_copy(ent_hbm.at[i_v.at[pl.ds(0, 128)]],
                              t_v.at[pl.ds(0, 128), :], sem0).wait()
        pltpu.make_async_copy(ent_hbm.at[i_v.at[pl.ds(128, NEG - 128)]],
                              t_v.at[pl.ds(128, NEG - 128), :], sem1).wait()

    masks = {s: (iota & s) == 0 for s in (1, 2, 4, 8)}
    rollm = {s: (iota - s) % L for s in (1, 2, 4, 8)}
    rollp = {s: (iota + s) % L for s in (1, 2, 4, 8)}

    def combine(a, bb, s):
        rb = bb.at[rollm[s]].get(mode="promise_in_bounds")
        ra = a.at[rollp[s]].get(mode="promise_in_bounds")
        return jnp.where(masks[s], a, rb) + jnp.where(masks[s], ra, bb)

    def compute_b(b, t_v):
        hr = [hr_v[pl.ds(b * DIM + s * L, L)] for s in range(DIM // L)]
        brow = b & 7

        def g_body(g, carry):
            off = lax.min(g * L, NEG - L)
            parts = []
            for k in _BITREV:
                row = off + k
                p01 = (jnp.abs(hr[0] - t_v[row, pl.ds(0, L)])
                       + jnp.abs(hr[1] - t_v[row, pl.ds(L, L)]))
                p23 = (jnp.abs(hr[2] - t_v[row, pl.ds(2 * L, L)])
                       + jnp.abs(hr[3] - t_v[row, pl.ds(3 * L, L)]))
                parts.append(p01 + p23)
            w = [combine(parts[2 * i], parts[2 * i + 1], 8) for i in range(8)]
            x = [combine(w[2 * i], w[2 * i + 1], 4) for i in range(4)]
            y = [combine(x[2 * i], x[2 * i + 1], 2) for i in range(2)]
            z = combine(y[0], y[1], 1)
            out8_v[brow, pl.ds(off, L)] = GAMMA - z
            return carry

        lax.fori_loop(0, len(_GROUP_OFFS), g_body, 0)

    start_tail(0, i0_v, t0_v)

    def b_body(i, carry):
        b = i * 2
        start_tail(b + 1, i1_v, t1_v)
        wait_tail(i0_v, t0_v)
        compute_b(b, t0_v)

        @pl.when(b + 2 < BPW)
        def _():
            start_tail(b + 2, i0_v, t0_v)
        wait_tail(i1_v, t1_v)
        compute_b(b + 1, t1_v)

        @pl.when((b & 7) == 6)
        def _():
            start = pl.multiple_of(base + b - 6, 8)
            pltpu.sync_copy(out8_v, out_hbm.at[pl.ds(start, 8), :])
        return carry

    lax.fori_loop(0, BPW // 2, b_body, 0)


@jax.jit
def _kge_score(positive_sample, negative_sample, entity_embedding,
               relation_embedding):
    hcol = positive_sample[:, 0]
    rcol = positive_sample[:, 1]
    ent2 = jnp.reshape(_to_fused(entity_embedding), (-1, DIM))
    rel2 = jnp.reshape(_to_fused(relation_embedding), (-1, DIM))
    mesh = plsc.VectorSubcoreMesh(core_axis_name="c", subcore_axis_name="s")
    run = functools.partial(
        pl.kernel,
        out_type=jax.ShapeDtypeStruct((BATCH, NEG), jnp.float32),
        mesh=mesh,
        compiler_params=pltpu.CompilerParams(
            needs_layout_passes=False, use_tc_tiling_on_sc=False),
        scratch_types=[
            pltpu.VMEM((BPW,), jnp.int32),
            pltpu.VMEM((BPW,), jnp.int32),
            pltpu.VMEM((BPW,), jnp.int32),
            pltpu.VMEM((BPW,), jnp.int32),
            pltpu.VMEM((BPW * DIM,), jnp.float32),
            pltpu.VMEM((BPW, NEG), jnp.int32),
            pltpu.VMEM((NEG,), jnp.int32),
            pltpu.VMEM((NEG,), jnp.int32),
            pltpu.VMEM((NEG, DIM), jnp.float32),
            pltpu.VMEM((NEG, DIM), jnp.float32),
            pltpu.VMEM((8, NEG), jnp.float32),
            pltpu.SemaphoreType.DMA,
            pltpu.SemaphoreType.DMA,
            pltpu.SemaphoreType.DMA,
            pltpu.SemaphoreType.DMA,
        ],
    )(_score_kernel)
    return run(hcol, rcol, negative_sample, ent2, rel2)


def kernel(positive_sample, negative_sample, entity_embedding,
           relation_embedding):
    return _kge_score(positive_sample, negative_sample, entity_embedding,
                      relation_embedding)


# speedup vs baseline: 2.0685x; 1.0290x over previous
import functools

import jax
import jax.numpy as jnp
from jax import lax
from jax.experimental import pallas as pl
from jax.experimental.pallas import tpu as pltpu
from jax.experimental.pallas import tpu_sc as plsc

GAMMA = 12.0
BATCH = 4096
NEG = 200
DIM = 64
FUSED = 128
EBLK = 32768
HALF = EBLK // 2

_info = plsc.get_sparse_core_info()
NC, NS, L = _info.num_cores, _info.num_subcores, _info.num_lanes
NW = NC * NS
BPW = BATCH // NW

_GROUP_OFFS = list(range(0, NEG - L, L)) + [NEG - L]

_BITREV = [0, 8, 4, 12, 2, 10, 6, 14, 1, 9, 5, 13, 3, 11, 7, 15]


def _repack_kernel(in_ref, out_ref):
    x = in_ref[...]
    out_ref[:, 0:DIM] = x[:, 0:HALF].T
    out_ref[:, DIM:FUSED] = x[:, HALF:EBLK].T


def _to_fused(table):
    n = table.shape[0]
    nblk = pl.cdiv(n, EBLK)
    return pl.pallas_call(
        _repack_kernel,
        grid=(nblk,),
        in_specs=[pl.BlockSpec((DIM, EBLK), lambda i: (0, i))],
        out_specs=pl.BlockSpec((HALF, FUSED), lambda i: (i, 0)),
        out_shape=jax.ShapeDtypeStruct((nblk * HALF, FUSED), jnp.float32),
    )(table.T)


_SH = EBLK.bit_length() - 1


def _row64(e):
    return ((e >> _SH) << _SH) | ((e & (HALF - 1)) << 1) | ((e >> (_SH - 1)) & 1)


def _score_kernel(hcol_hbm, rcol_hbm, neg_hbm, ent_hbm, rel_hbm, out_hbm,
                  hidx_v, ridx_v, hsh_v, rsh_v, hr_v, nidx_v,
                  i0_v, i1_v, t0_v, t1_v, out8_v, sem0, sem1, semh, semr):
    wid = lax.axis_index("s") * NC + lax.axis_index("c")
    base = wid * BPW
    iota = lax.iota(jnp.int32, L)

    pltpu.sync_copy(hcol_hbm.at[pl.ds(base, BPW)], hidx_v)
    pltpu.sync_copy(rcol_hbm.at[pl.ds(base, BPW)], ridx_v)
    pltpu.sync_copy(neg_hbm.at[pl.ds(base, BPW), :], nidx_v)

    for i in range(BPW // L):
        sl = pl.ds(i * L, L)
        hsh_v[sl] = _row64(hidx_v[sl])
        rsh_v[sl] = _row64(ridx_v[sl])

    ch = pltpu.async_copy(ent_hbm.at[hsh_v], t0_v.at[pl.ds(0, BPW), :], semh)
    cr = pltpu.async_copy(rel_hbm.at[rsh_v], t1_v.at[pl.ds(0, BPW), :], semr)
    ch.wait()
    cr.wait()

    def hr_body(b, carry):
        for s in range(DIM // L):
            sl = pl.ds(s * L, L)
            hr_v[pl.ds(b * DIM + s * L, L)] = t0_v[b, sl] + t1_v[b, sl]
        return carry
    lax.fori_loop(0, BPW, hr_body, 0)

    def prep_idx(b, i_v):
        for off in _GROUP_OFFS:
            sl = pl.ds(off, L)
            i_v[sl] = _row64(nidx_v[b, sl])

    def start_tail(b, i_v, t_v):
        prep_idx(b, i_v)
        pltpu.async_copy(ent_hbm.at[i_v.at[pl.ds(0, 128)]],
                         t_v.at[pl.ds(0, 128), :], sem0)
        pltpu.async_copy(ent_hbm.at[i_v.at[pl.ds(128, NEG - 128)]],
                         t_v.at[pl.ds(128, NEG - 128), :], sem1)

    def wait_tail(i_v, t_v):
        pltpu.make_async_copy(ent_hbm.at[i_v.at[pl.ds(0, 128)]],
                              t_v.at[pl.ds(0, 128), :], sem0).wait()
        pltpu.make_async_copy(ent_hbm.at[i_v.at[pl.ds(128, NEG - 128)]],
                              t_v.at[pl.ds(128, NEG - 128), :], sem1).wait()

    masks = {s: (iota & s) == 0 for s in (1, 2, 4, 8)}
    rollm = {s: (iota - s) % L for s in (1, 2, 4, 8)}
    rollp = {s: (iota + s) % L for s in (1, 2, 4, 8)}

    def combine(a, bb, s):
        rb = bb.at[rollm[s]].get(mode="promise_in_bounds")
        ra = a.at[rollp[s]].get(mode="promise_in_bounds")
        return jnp.where(masks[s], a, rb) + jnp.where(masks[s], ra, bb)

    def compute_b(b, t_v):
        hr = [hr_v[pl.ds(b * DIM + s * L, L)] for s in range(DIM // L)]
        brow = b & 7

        def g_body(g, carry):
            off = lax.min(g * L, NEG - L)
            parts = []
            for k in _BITREV:
                row = off + k
                p01 = (jnp.abs(hr[0] - t_v[row, pl.ds(0, L)])
                       + jnp.abs(hr[1] - t_v[row, pl.ds(L, L)]))
                p23 = (jnp.abs(hr[2] - t_v[row, pl.ds(2 * L, L)])
                       + jnp.abs(hr[3] - t_v[row, pl.ds(3 * L, L)]))
                parts.append(p01 + p23)
            w = [combine(parts[2 * i], parts[2 * i + 1], 8) for i in range(8)]
            x = [combine(w[2 * i], w[2 * i + 1], 4) for i in range(4)]
            y = [combine(x[2 * i], x[2 * i + 1], 2) for i in range(2)]
            z = combine(y[0], y[1], 1)
            out8_v[brow, pl.ds(off, L)] = GAMMA - z
            return carry

        lax.fori_loop(0, len(_GROUP_OFFS), g_body, 0)

    start_tail(0, i0_v, t0_v)

    def b_body(i, carry):
        b = i * 2
        start_tail(b + 1, i1_v, t1_v)
        wait_tail(i0_v, t0_v)
        compute_b(b, t0_v)

        @pl.when(b + 2 < BPW)
        def _():
            start_tail(b + 2, i0_v, t0_v)
        wait_tail(i1_v, t1_v)
        compute_b(b + 1, t1_v)

        @pl.when((b & 7) == 6)
        def _():
            start = pl.multiple_of(base + b - 6, 8)
            pltpu.sync_copy(out8_v, out_hbm.at[pl.ds(start, 8), :])
        return carry

    lax.fori_loop(0, BPW // 2, b_body, 0)


@jax.jit
def _kge_score(positive_sample, negative_sample, entity_embedding,
               relation_embedding):
    hcol = positive_sample[:, 0]
    rcol = positive_sample[:, 1]
    ent2 = jnp.reshape(_to_fused(entity_embedding), (-1, DIM))
    rel2 = jnp.reshape(_to_fused(relation_embedding), (-1, DIM))
    mesh = plsc.VectorSubcoreMesh(core_axis_name="c", subcore_axis_name="s")
    run = functools.partial(
        pl.kernel,
        out_type=jax.ShapeDtypeStruct((BATCH, NEG), jnp.float32),
        mesh=mesh,
        compiler_params=pltpu.CompilerParams(
            needs_layout_passes=False, use_tc_tiling_on_sc=False),
        scratch_types=[
            pltpu.VMEM((BPW,), jnp.int32),
            pltpu.VMEM((BPW,), jnp.int32),
            pltpu.VMEM((BPW,), jnp.int32),
            pltpu.VMEM((BPW,), jnp.int32),
            pltpu.VMEM((BPW * DIM,), jnp.float32),
            pltpu.VMEM((BPW, NEG), jnp.int32),
            pltpu.VMEM((NEG,), jnp.int32),
            pltpu.VMEM((NEG,), jnp.int32),
            pltpu.VMEM((NEG, DIM), jnp.float32),
            pltpu.VMEM((NEG, DIM), jnp.float32),
            pltpu.VMEM((8, NEG), jnp.float32),
            pltpu.SemaphoreType.DMA,
            pltpu.SemaphoreType.DMA,
            pltpu.SemaphoreType.DMA,
            pltpu.SemaphoreType.DMA,
        ],
    )(_score_kernel)
    return run(hcol, rcol, negative_sample, ent2, rel2)


def kernel(positive_sample, negative_sample, entity_embedding,
           relation_embedding):
    return _kge_score(positive_sample, negative_sample, entity_embedding,
                      relation_embedding)
